# Initial kernel scaffold; baseline (speedup 1.0000x reference)
import functools

import jax
import jax.numpy as jnp
from jax import lax
from jax.experimental import pallas as pl
from jax.experimental.pallas import tpu as pltpu
from jax.experimental.pallas import tpu_sc as plsc

SEQ = 200
D = 32
BATCH = 4096
SCALE = float(32.0 ** 0.5)

NC = 2
NS = 16
NW = NC * NS

ROWS = BATCH * SEQ
PER_W = ROWS // NW
CHUNK = 800
N_CHUNKS = PER_W // CHUNK
POS_REP = CHUNK // SEQ


def _body(idx_hbm, tok_hbm, pos_hbm, out_hbm, idx_v, rows_v, pos_v, sem):
    wid = lax.axis_index("s") * NC + lax.axis_index("c")
    base_w = wid * PER_W

    for i in range(POS_REP):
        pltpu.sync_copy(pos_hbm, pos_v.at[pl.ds(i * SEQ, SEQ)])

    def chunk_body(g, carry):
        base = base_w + g * CHUNK
        pltpu.sync_copy(idx_hbm.at[pl.ds(base, CHUNK)], idx_v)
        pltpu.async_copy(tok_hbm.at[idx_v], rows_v, sem).wait()

        def row_body(r, c):
            for h in range(D // 16):
                sl = pl.ds(h * 16, 16)
                rows_v[r, sl] = rows_v[r, sl] * SCALE + pos_v[r, sl]
            return c

        lax.fori_loop(0, CHUNK, row_body, 0)
        pltpu.sync_copy(rows_v, out_hbm.at[pl.ds(base, CHUNK)])
        return carry

    lax.fori_loop(0, N_CHUNKS, chunk_body, 0)


@jax.jit
def _embed(flat_idx, token_table, pos_table):
    mesh = plsc.VectorSubcoreMesh(core_axis_name="c", subcore_axis_name="s")
    return pl.kernel(
        _body,
        out_type=jax.ShapeDtypeStruct((ROWS, D), jnp.float32),
        mesh=mesh,
        scratch_types=[
            pltpu.VMEM((CHUNK,), jnp.int32),
            pltpu.VMEM((CHUNK, D), jnp.float32),
            pltpu.VMEM((CHUNK, D), jnp.float32),
            pltpu.SemaphoreType.DMA,
        ],
    )(flat_idx, token_table, pos_table)


def kernel(inputs, token_table, pos_table):
    b, s = inputs.shape
    flat_idx = inputs.reshape(-1).astype(jnp.int32)
    out = _embed(flat_idx, token_table, pos_table)
    return out.reshape(b, s, D)


# speedup vs baseline: 1.2614x; 1.2614x over previous
import functools

import jax
import jax.numpy as jnp
from jax import lax
from jax.experimental import pallas as pl
from jax.experimental.pallas import tpu as pltpu
from jax.experimental.pallas import tpu_sc as plsc

SEQ = 200
D = 32
BATCH = 4096
SCALE = float(32.0 ** 0.5)

NC = 2
NS = 16
NW = NC * NS

ROWS = BATCH * SEQ
PER_W = ROWS // NW
CHUNK = 800
N_CHUNKS = PER_W // CHUNK
POS_REP = CHUNK // SEQ


def _body(idx_hbm, tok_hbm, pos_hbm, out_hbm, idx_v, rows_v, pos_v, sem):
    wid = lax.axis_index("s") * NC + lax.axis_index("c")
    base_w = wid * PER_W

    for i in range(POS_REP):
        pltpu.sync_copy(pos_hbm, pos_v.at[pl.ds(i * SEQ, SEQ)])

    def chunk_body(g, carry):
        base = base_w + g * CHUNK
        pltpu.sync_copy(idx_hbm.at[pl.ds(base, CHUNK)], idx_v)
        pltpu.async_copy(tok_hbm.at[idx_v], rows_v, sem).wait()

        def row_body(r, c):
            for h in range(D // 16):
                sl = pl.ds(h * 16, 16)
                rows_v[r, sl] = rows_v[r, sl] * SCALE + pos_v[r, sl]
            return c

        lax.fori_loop(0, CHUNK, row_body, 0)
        pltpu.sync_copy(rows_v, out_hbm.at[pl.ds(base, CHUNK)])
        return carry

    lax.fori_loop(0, N_CHUNKS, chunk_body, 0)


@jax.jit
def _embed(flat_idx, token_table, pos_table):
    mesh = plsc.VectorSubcoreMesh(core_axis_name="c", subcore_axis_name="s")
    return pl.kernel(
        _body,
        out_type=jax.ShapeDtypeStruct((ROWS, D), jnp.float32),
        mesh=mesh,
        compiler_params=pltpu.CompilerParams(use_tc_tiling_on_sc=False),
        scratch_types=[
            pltpu.VMEM((CHUNK,), jnp.int32),
            pltpu.VMEM((CHUNK, D), jnp.float32),
            pltpu.VMEM((CHUNK, D), jnp.float32),
            pltpu.SemaphoreType.DMA,
        ],
    )(flat_idx, token_table, pos_table)


def kernel(inputs, token_table, pos_table):
    b, s = inputs.shape
    flat_idx = inputs.reshape(-1).astype(jnp.int32)
    out = _embed(flat_idx, token_table, pos_table)
    return out.reshape(b, s, D)


# speedup vs baseline: 1.5675x; 1.2427x over previous
import jax
import jax.numpy as jnp
from jax import lax
from jax.experimental import pallas as pl
from jax.experimental.pallas import tpu as pltpu
from jax.experimental.pallas import tpu_sc as plsc

SEQ = 200
D = 32
BATCH = 4096
SCALE = float(32.0 ** 0.5)

NC = 2
NS = 16
NW = NC * NS

ROWS = BATCH * SEQ
CHUNK = 512
CPS = BATCH // CHUNK
N_CHUNKS = ROWS // CHUNK
PER_W = N_CHUNKS // NW
RING = 4


def _body(idx_hbm, tok_hbm, pos_hbm, out_hbm, pos_v, *bufs):
    idx_v = bufs[0:RING]
    rows_v = bufs[RING:2 * RING]
    isem = bufs[2 * RING:3 * RING]
    gsem = bufs[3 * RING:4 * RING]
    osem = bufs[4 * RING:5 * RING]

    wid = lax.axis_index("s") * NC + lax.axis_index("c")
    base = wid * PER_W

    pltpu.sync_copy(pos_hbm, pos_v)

    def chunk_sb(c):
        t = base + c
        return t // CPS, (t % CPS) * CHUNK

    def idx_copy(c):
        s, b0 = chunk_sb(c)
        r = c % RING
        return pltpu.make_async_copy(
            idx_hbm.at[s, pl.ds(b0, CHUNK)], idx_v[r], isem[r])

    def gather_copy(c):
        r = c % RING
        return pltpu.make_async_copy(
            tok_hbm.at[idx_v[r]], rows_v[r], gsem[r])

    def out_copy(c):
        s, b0 = chunk_sb(c)
        r = c % RING
        return pltpu.make_async_copy(
            rows_v[r], out_hbm.at[s, pl.ds(b0, CHUNK)], osem[r])

    def compute(c):
        s, _b0 = chunk_sb(c)
        r = c % RING
        buf = rows_v[r]
        plo = pos_v[s, pl.ds(0, 16)]
        phi = pos_v[s, pl.ds(16, 16)]

        def row4(rb, carry):
            r0 = rb * 4
            for rr in range(4):
                buf[r0 + rr, pl.ds(0, 16)] = (
                    buf[r0 + rr, pl.ds(0, 16)] * SCALE + plo)
                buf[r0 + rr, pl.ds(16, 16)] = (
                    buf[r0 + rr, pl.ds(16, 16)] * SCALE + phi)
            return carry

        lax.fori_loop(0, CHUNK // 4, row4, 0)

    for c in range(RING):
        idx_copy(c).start()
    for c in range(RING - 1):
        idx_copy(c).wait()
        gather_copy(c).start()

    for c in range(PER_W):
        gather_copy(c).wait()
        compute(c)
        out_copy(c).start()
        if c + RING < PER_W:
            idx_copy(c + RING).start()
        if c + RING - 1 < PER_W:
            if c >= 1:
                out_copy(c - 1).wait()
            idx_copy(c + RING - 1).wait()
            gather_copy(c + RING - 1).start()

    for c in range(PER_W - RING, PER_W):
        out_copy(c).wait()


@jax.jit
def _embed(idx_t, token_table, pos_table):
    mesh = plsc.VectorSubcoreMesh(core_axis_name="c", subcore_axis_name="s")
    return pl.kernel(
        _body,
        out_type=jax.ShapeDtypeStruct((SEQ, BATCH, D), jnp.float32),
        mesh=mesh,
        compiler_params=pltpu.CompilerParams(use_tc_tiling_on_sc=False),
        scratch_types=(
            [pltpu.VMEM((SEQ, D), jnp.float32)]
            + [pltpu.VMEM((CHUNK,), jnp.int32) for _ in range(RING)]
            + [pltpu.VMEM((CHUNK, D), jnp.float32) for _ in range(RING)]
            + [pltpu.SemaphoreType.DMA for _ in range(3 * RING)]
        ),
    )(idx_t, token_table, pos_table)


def kernel(inputs, token_table, pos_table):
    b, s = inputs.shape
    idx_t = jnp.transpose(inputs).astype(jnp.int32)
    out_sm = _embed(idx_t, token_table, pos_table)
    return jnp.transpose(out_sm, (1, 0, 2))
